# baseline (speedup 1.0000x reference)
import functools

import jax
import jax.numpy as jnp
from jax import lax
from jax.experimental import pallas as pl
from jax.experimental.pallas import tpu as pltpu
from jax.experimental.pallas import tpu_sc as plsc

N = 10000
E = 320000
IN_DIM = 128
HID = 64

NC = 2
NS = 16
NW = NC * NS
L = 16
NP = 10240
RPT = NP // NS
EPW = E // NW
CH = 80
NCH = EPW // CH
HIDP = 128


@functools.cache
def _sc_kernels():
    mesh = plsc.VectorSubcoreMesh(core_axis_name="c", subcore_axis_name="s",
                                  num_cores=NC, num_subcores=NS)
    params = pltpu.CompilerParams(needs_layout_passes=False)
    deg = functools.partial(
        pl.kernel,
        out_type=jax.ShapeDtypeStruct((NW, NP), jnp.float32),
        mesh=mesh,
        compiler_params=params,
        scratch_types=[
            pltpu.VMEM((EPW,), jnp.int32),
            pltpu.VMEM((NP,), jnp.float32),
        ],
    )(_deg_body)
    mp = functools.partial(
        pl.kernel,
        out_type=jax.ShapeDtypeStruct((NC, NP, HIDP), jnp.float32),
        mesh=mesh,
        compiler_params=params,
        scratch_types=[
            pltpu.VMEM((EPW,), jnp.int32),
            pltpu.VMEM((NCH, CH), jnp.int32),
            pltpu.VMEM((CH, HIDP), jnp.float32),
            pltpu.VMEM((CH, HIDP), jnp.float32),
            pltpu.VMEM_SHARED((NP, HIDP), jnp.float32),
            pltpu.SemaphoreType.DMA,
            pltpu.SemaphoreType.DMA,
            pltpu.SemaphoreType.DMA,
            pltpu.SemaphoreType.DMA,
        ],
    )(_mp_body)
    eg = functools.partial(
        pl.kernel,
        out_type=jax.ShapeDtypeStruct((E // 2, HIDP), jnp.int32),
        mesh=mesh,
        compiler_params=params,
        scratch_types=[
            pltpu.VMEM((EPW,), jnp.int32),
            pltpu.VMEM((EPW,), jnp.int32),
            pltpu.VMEM((CH, HIDP), jnp.float32),
            pltpu.VMEM((CH, HIDP), jnp.float32),
            pltpu.VMEM((CH, HIDP), jnp.float32),
            pltpu.VMEM((CH, HIDP), jnp.float32),
            pltpu.VMEM((CH // 2, HIDP), jnp.int32),
            pltpu.VMEM((CH // 2, HIDP), jnp.int32),
            pltpu.SemaphoreType.DMA,
            pltpu.SemaphoreType.DMA,
            pltpu.SemaphoreType.DMA,
            pltpu.SemaphoreType.DMA,
            pltpu.SemaphoreType.DMA,
            pltpu.SemaphoreType.DMA,
        ],
    )(_eg_body)
    return deg, mp, eg


def _deg_body(dst_hbm, out_hbm, idx_v, deg_v):
    cid = lax.axis_index("c")
    sid = lax.axis_index("s")
    wid = sid * NC + cid
    zero = jnp.zeros((L,), jnp.float32)

    def _z(i, c):
        deg_v[pl.ds(i * L, L)] = zero
        return c
    lax.fori_loop(0, NP // L, _z, 0)

    pltpu.sync_copy(dst_hbm.at[pl.ds(wid * EPW, EPW)], idx_v)
    one = jnp.ones((L,), jnp.float32)

    def _a(i, c):
        plsc.addupdate_scatter(deg_v, [idx_v[pl.ds(i * L, L)]], one)
        return c
    lax.fori_loop(0, EPW // L, _a, 0)

    pltpu.sync_copy(deg_v, out_hbm.at[wid])


def _mp_body(src_hbm, dst_hbm, y_hbm, out_hbm, sidx1, didx2, rows0, rows1,
             acc, gs0, gs1, ss0, ss1):
    cid = lax.axis_index("c")
    sid = lax.axis_index("s")
    wid = sid * NC + cid

    zero = jnp.zeros((L,), jnp.float32)

    def _zr(r, c):
        for c4 in range(HIDP // L):
            rows0[r, pl.ds(c4 * L, L)] = zero
        return c
    lax.fori_loop(0, CH, _zr, 0)

    def _z(j, c):
        pltpu.sync_copy(rows0, acc.at[pl.ds(sid * RPT + j * CH, CH), :])
        return c
    lax.fori_loop(0, RPT // CH, _z, 0)

    pltpu.sync_copy(src_hbm.at[pl.ds(wid * EPW, EPW)], sidx1)
    pltpu.sync_copy(dst_hbm.at[wid], didx2)
    plsc.subcore_barrier()

    def _wait_g(k, rows, gs):
        pltpu.make_async_copy(
            y_hbm.at[sidx1.at[pl.ds(k * CH, CH)]], rows, gs).wait()

    def _fire_g(k, rows, gs):
        pltpu.async_copy(y_hbm.at[sidx1.at[pl.ds(k * CH, CH)]], rows, gs)

    def _fire_s(k, rows, ss):
        pltpu.async_copy(rows, acc.at[didx2.at[k]], ss, add=True)

    def _wait_s(k, rows, ss):
        pltpu.make_async_copy(rows, acc.at[didx2.at[k]], ss).wait()

    _fire_g(0, rows0, gs0)

    def _pair(p, c):
        k0 = 2 * p
        k1 = k0 + 1
        _wait_g(k0, rows0, gs0)

        @pl.when(p > 0)
        def _():
            _wait_s(k0 - 1, rows1, ss1)
        _fire_g(k1, rows1, gs1)
        _fire_s(k0, rows0, ss0)
        _wait_g(k1, rows1, gs1)
        _wait_s(k0, rows0, ss0)

        @pl.when(k0 + 2 < NCH)
        def _():
            _fire_g(k0 + 2, rows0, gs0)
        _fire_s(k1, rows1, ss1)
        return c
    lax.fori_loop(0, NCH // 2, _pair, 0)

    _wait_g(NCH - 1, rows0, gs0)
    _wait_s(NCH - 2, rows1, ss1)
    _fire_s(NCH - 1, rows0, ss0)
    _wait_s(NCH - 1, rows0, ss0)

    plsc.subcore_barrier()
    pltpu.sync_copy(acc.at[pl.ds(sid * RPT, RPT), :],
                    out_hbm.at[cid, pl.ds(sid * RPT, RPT), :])


def _eg_body(e0_hbm, e1_hbm, h_hbm, uv_hbm, i0, i1,
             ru0, ru1, rv0, rv1, rp0, rp1, su0, su1, sv0, sv1, sw0, sw1):
    cid = lax.axis_index("c")
    sid = lax.axis_index("s")
    wid = sid * NC + cid
    base = wid * EPW
    pltpu.sync_copy(e0_hbm.at[pl.ds(base, EPW)], i0)
    pltpu.sync_copy(e1_hbm.at[pl.ds(base, EPW)], i1)

    def _gath(k, ru, su, rv, sv):
        pltpu.async_copy(h_hbm.at[i0.at[pl.ds(k * CH, CH)]], ru, su)
        pltpu.async_copy(h_hbm.at[i1.at[pl.ds(k * CH, CH)]], rv, sv)

    def _wait(k, ru, su, rv, sv):
        pltpu.make_async_copy(h_hbm.at[i0.at[pl.ds(k * CH, CH)]], ru, su).wait()
        pltpu.make_async_copy(h_hbm.at[i1.at[pl.ds(k * CH, CH)]], rv, sv).wait()

    def _bf16_bits(x):
        xi = plsc.bitcast(x, jnp.int32)
        return xi + 0x7FFF + (lax.shift_right_logical(xi, 16) & 1)

    def _pack(ru, rv, rp):
        def _rows8(r8, c):
            for dr in range(8):
                r = r8 * 8 + dr
                half = (r % 2) * HID
                for c4 in range(HID // L):
                    ub = lax.shift_right_logical(
                        _bf16_bits(ru[r, pl.ds(c4 * L, L)]), 16)
                    vb = _bf16_bits(rv[r, pl.ds(c4 * L, L)]) & jnp.int32(
                        -65536)
                    rp[r // 2, pl.ds(half + c4 * L, L)] = ub | vb

            return c
        lax.fori_loop(0, CH // 8, _rows8, 0)

    def _fire_w(k, rp, sw):
        off = pl.multiple_of((base + k * CH) // 2, 8)
        pltpu.async_copy(rp, uv_hbm.at[pl.ds(off, CH // 2), :], sw)

    def _wait_w(k, rp, sw):
        off = pl.multiple_of((base + k * CH) // 2, 8)
        pltpu.make_async_copy(rp, uv_hbm.at[pl.ds(off, CH // 2), :], sw).wait()

    _gath(0, ru0, su0, rv0, sv0)

    def _pair(p, c):
        k0 = 2 * p
        k1 = k0 + 1
        _wait(k0, ru0, su0, rv0, sv0)
        _gath(k1, ru1, su1, rv1, sv1)

        @pl.when(p > 0)
        def _():
            _wait_w(k0 - 2, rp0, sw0)
        _pack(ru0, rv0, rp0)
        _fire_w(k0, rp0, sw0)

        _wait(k1, ru1, su1, rv1, sv1)

        @pl.when(k0 + 2 < NCH)
        def _():
            _gath(k0 + 2, ru0, su0, rv0, sv0)

        @pl.when(p > 0)
        def _():
            _wait_w(k1 - 2, rp1, sw1)
        _pack(ru1, rv1, rp1)
        _fire_w(k1, rp1, sw1)
        return c
    lax.fori_loop(0, NCH // 2, _pair, 0)

    _wait(NCH - 1, ru0, su0, rv0, sv0)
    _wait_w(NCH - 3, rp0, sw0)
    _pack(ru0, rv0, rp0)
    _fire_w(NCH - 1, rp0, sw0)
    _wait_w(NCH - 2, rp1, sw1)
    _wait_w(NCH - 1, rp0, sw0)


def _dis_col(degp):
    ones = jnp.ones((NW, 1), jnp.float32)
    deg = lax.dot_general(degp, ones, (((0,), (0,)), ((), ())),
                          preferred_element_type=jnp.float32)
    return lax.rsqrt(deg[:N] + 1.0)


def _pad_cols(a):
    return jnp.concatenate(
        [a, jnp.zeros((a.shape[0], HIDP - HID), jnp.float32)], axis=1)


def _prep1_body(degp_ref, x_ref, w_ref, y_ref):
    dis = _dis_col(degp_ref[...])
    y = jnp.dot(x_ref[...], w_ref[...],
                preferred_element_type=jnp.float32) * dis
    y_ref[...] = _pad_cols(y)


_prep1 = pl.pallas_call(
    _prep1_body,
    out_shape=jax.ShapeDtypeStruct((N, HIDP), jnp.float32),
)


def _mid_body(degp_ref, accp_ref, y1_ref, b1_ref, w2_ref, y2_ref):
    dis = _dis_col(degp_ref[...])
    acc = accp_ref[0, :N, 0:HID] + accp_ref[1, :N, 0:HID]
    h1 = jnp.maximum((acc + y1_ref[:, 0:HID]) * dis + b1_ref[...], 0.0)
    y2 = jnp.dot(h1, w2_ref[...], preferred_element_type=jnp.float32) * dis
    y2_ref[...] = _pad_cols(y2)


_mid = pl.pallas_call(
    _mid_body,
    out_shape=jax.ShapeDtypeStruct((N, HIDP), jnp.float32),
)


def _fin_body(degp_ref, accp_ref, y2_ref, b2_ref, h2_ref):
    dis = _dis_col(degp_ref[...])
    acc = accp_ref[0, :N, 0:HID] + accp_ref[1, :N, 0:HID]
    h2 = jnp.maximum((acc + y2_ref[:, 0:HID]) * dis + b2_ref[...], 0.0)
    h2_ref[...] = _pad_cols(h2)


_fin = pl.pallas_call(
    _fin_body,
    out_shape=jax.ShapeDtypeStruct((N, HIDP), jnp.float32),
)


BE = 8000


def _mlp_body(uv_ref, wa_ref, wb_ref, wc_ref, wd_ref, b1_ref, w2_ref,
              b2_ref, o_ref):
    bf = jnp.bfloat16
    z32 = uv_ref[...]
    uf = lax.bitcast_convert_type(jnp.left_shift(z32, 16), jnp.float32)
    vf = lax.bitcast_convert_type(
        jnp.bitwise_and(z32, jnp.int32(-65536)), jnp.float32)
    t = jnp.dot(uf.astype(bf), wa_ref[...].astype(bf),
                preferred_element_type=jnp.float32)
    t = t + jnp.dot(vf.astype(bf), wb_ref[...].astype(bf),
                    preferred_element_type=jnp.float32)
    t = t + jnp.dot(jnp.abs(uf - vf).astype(bf), wc_ref[...].astype(bf),
                    preferred_element_type=jnp.float32)
    t = t + jnp.dot((uf * vf).astype(bf), wd_ref[...].astype(bf),
                    preferred_element_type=jnp.float32)
    t = jnp.maximum(t + b1_ref[...], 0.0)
    o2 = jnp.dot(t.astype(bf), w2_ref[...].astype(bf),
                 preferred_element_type=jnp.float32)
    o_ref[...] = o2 + b2_ref[...]


BE2 = BE // 2

_mlp = pl.pallas_call(
    _mlp_body,
    grid=(E // BE,),
    in_specs=[
        pl.BlockSpec((BE2, HIDP), lambda i: (i, 0)),
        pl.BlockSpec((HIDP, HIDP), lambda i: (0, 0)),
        pl.BlockSpec((HIDP, HIDP), lambda i: (0, 0)),
        pl.BlockSpec((HIDP, HIDP), lambda i: (0, 0)),
        pl.BlockSpec((HIDP, HIDP), lambda i: (0, 0)),
        pl.BlockSpec((HIDP,), lambda i: (0,)),
        pl.BlockSpec((HIDP, 2), lambda i: (0, 0)),
        pl.BlockSpec((1,), lambda i: (0,)),
    ],
    out_specs=pl.BlockSpec((BE2, 2), lambda i: (i, 0)),
    out_shape=jax.ShapeDtypeStruct((E // 2, 2), jnp.float32),
)


def kernel(x, graph_edge_index, edge_pairs, W_gcn1, b_gcn1, W_gcn2, b_gcn2,
           W_mlp1, b_mlp1, W_mlp2, b_mlp2):
    _deg_sc, _mp_sc, _eg_sc = _sc_kernels()
    src = graph_edge_index[0]
    dst = graph_edge_index[1]
    dst2 = dst.reshape(NW, NCH, CH)

    degp = _deg_sc(dst)
    y1 = _prep1(degp, x, W_gcn1)
    accp1 = _mp_sc(src, dst2, y1)
    y2 = _mid(degp, accp1, y1, b_gcn1, W_gcn2)
    accp2 = _mp_sc(src, dst2, y2)
    h2 = _fin(degp, accp2, y2, b_gcn2)
    uv = _eg_sc(edge_pairs[0], edge_pairs[1], h2)

    def _bd(w, cols):
        z = jnp.zeros((HID, cols), jnp.float32)
        top = jnp.concatenate([w, z], axis=1)
        bot = jnp.concatenate([z, w], axis=1)
        return jnp.concatenate([top, bot], axis=0)

    wa = _bd(W_mlp1[0:HID], HID)
    wb = _bd(W_mlp1[HID:2 * HID], HID)
    wc = _bd(W_mlp1[2 * HID:3 * HID], HID)
    wd = _bd(W_mlp1[3 * HID:4 * HID], HID)
    b1b = jnp.concatenate([b_mlp1, b_mlp1])
    w2b = _bd(W_mlp2, 1)
    return _mlp(uv, wa, wb, wc, wd, b1b, w2b, b_mlp2).reshape(-1)


# speedup vs baseline: 1.0661x; 1.0661x over previous
import functools

import jax
import jax.numpy as jnp
from jax import lax
from jax.experimental import pallas as pl
from jax.experimental.pallas import tpu as pltpu
from jax.experimental.pallas import tpu_sc as plsc

N = 10000
E = 320000
IN_DIM = 128
HID = 64

NC = 2
NS = 16
NW = NC * NS
L = 16
NP = 10240
RPT = NP // NS
EPW = E // NW
CH = 80
NCH = EPW // CH
HIDP = 128


@functools.cache
def _sc_kernels():
    mesh = plsc.VectorSubcoreMesh(core_axis_name="c", subcore_axis_name="s",
                                  num_cores=NC, num_subcores=NS)
    params = pltpu.CompilerParams(needs_layout_passes=False)
    deg = functools.partial(
        pl.kernel,
        out_type=jax.ShapeDtypeStruct((NW, NP), jnp.float32),
        mesh=mesh,
        compiler_params=params,
        scratch_types=[
            pltpu.VMEM((EPW,), jnp.int32),
            pltpu.VMEM((NP,), jnp.float32),
        ],
    )(_deg_body)
    mp = functools.partial(
        pl.kernel,
        out_type=jax.ShapeDtypeStruct((NC, NP, HIDP), jnp.float32),
        mesh=mesh,
        compiler_params=params,
        scratch_types=[
            pltpu.VMEM((EPW,), jnp.int32),
            pltpu.VMEM((NCH, CH), jnp.int32),
            pltpu.VMEM((CH, HIDP), jnp.float32),
            pltpu.VMEM((CH, HIDP), jnp.float32),
            pltpu.VMEM_SHARED((NP, HIDP), jnp.float32),
            pltpu.SemaphoreType.DMA,
            pltpu.SemaphoreType.DMA,
            pltpu.SemaphoreType.DMA,
            pltpu.SemaphoreType.DMA,
        ],
    )(_mp_body)
    eg = functools.partial(
        pl.kernel,
        out_type=jax.ShapeDtypeStruct((E, HIDP), jnp.float32),
        mesh=mesh,
        compiler_params=params,
        scratch_types=[
            pltpu.VMEM((EPW,), jnp.int32),
            pltpu.VMEM((EPW,), jnp.int32),
            pltpu.VMEM((CH, HIDP), jnp.float32),
            pltpu.VMEM((CH, HIDP), jnp.float32),
            pltpu.VMEM((CH, HIDP), jnp.float32),
            pltpu.VMEM((CH, HIDP), jnp.float32),
            pltpu.VMEM((CH, HIDP), jnp.float32),
            pltpu.VMEM((CH, HIDP), jnp.float32),
            pltpu.SemaphoreType.DMA,
            pltpu.SemaphoreType.DMA,
            pltpu.SemaphoreType.DMA,
            pltpu.SemaphoreType.DMA,
            pltpu.SemaphoreType.DMA,
            pltpu.SemaphoreType.DMA,
        ],
    )(_eg_body)
    return deg, mp, eg


def _deg_body(dst_hbm, out_hbm, idx_v, deg_v):
    cid = lax.axis_index("c")
    sid = lax.axis_index("s")
    wid = sid * NC + cid
    zero = jnp.zeros((L,), jnp.float32)

    def _z(i, c):
        deg_v[pl.ds(i * L, L)] = zero
        return c
    lax.fori_loop(0, NP // L, _z, 0)

    pltpu.sync_copy(dst_hbm.at[pl.ds(wid * EPW, EPW)], idx_v)
    one = jnp.ones((L,), jnp.float32)

    def _a(i, c):
        plsc.addupdate_scatter(deg_v, [idx_v[pl.ds(i * L, L)]], one)
        return c
    lax.fori_loop(0, EPW // L, _a, 0)

    pltpu.sync_copy(deg_v, out_hbm.at[wid])


def _mp_body(src_hbm, dst_hbm, y_hbm, out_hbm, sidx1, didx2, rows0, rows1,
             acc, gs0, gs1, ss0, ss1):
    cid = lax.axis_index("c")
    sid = lax.axis_index("s")
    wid = sid * NC + cid

    zero = jnp.zeros((L,), jnp.float32)

    def _zr(r, c):
        for c4 in range(HIDP // L):
            rows0[r, pl.ds(c4 * L, L)] = zero
        return c
    lax.fori_loop(0, CH, _zr, 0)

    def _z(j, c):
        pltpu.sync_copy(rows0, acc.at[pl.ds(sid * RPT + j * CH, CH), :])
        return c
    lax.fori_loop(0, RPT // CH, _z, 0)

    pltpu.sync_copy(src_hbm.at[pl.ds(wid * EPW, EPW)], sidx1)
    pltpu.sync_copy(dst_hbm.at[wid], didx2)
    plsc.subcore_barrier()

    def _wait_g(k, rows, gs):
        pltpu.make_async_copy(
            y_hbm.at[sidx1.at[pl.ds(k * CH, CH)]], rows, gs).wait()

    def _fire_g(k, rows, gs):
        pltpu.async_copy(y_hbm.at[sidx1.at[pl.ds(k * CH, CH)]], rows, gs)

    def _fire_s(k, rows, ss):
        pltpu.async_copy(rows, acc.at[didx2.at[k]], ss, add=True)

    def _wait_s(k, rows, ss):
        pltpu.make_async_copy(rows, acc.at[didx2.at[k]], ss).wait()

    _fire_g(0, rows0, gs0)

    def _pair(p, c):
        k0 = 2 * p
        k1 = k0 + 1
        _wait_g(k0, rows0, gs0)

        @pl.when(p > 0)
        def _():
            _wait_s(k0 - 1, rows1, ss1)
        _fire_g(k1, rows1, gs1)
        _fire_s(k0, rows0, ss0)
        _wait_g(k1, rows1, gs1)
        _wait_s(k0, rows0, ss0)

        @pl.when(k0 + 2 < NCH)
        def _():
            _fire_g(k0 + 2, rows0, gs0)
        _fire_s(k1, rows1, ss1)
        return c
    lax.fori_loop(0, NCH // 2, _pair, 0)

    _wait_g(NCH - 1, rows0, gs0)
    _wait_s(NCH - 2, rows1, ss1)
    _fire_s(NCH - 1, rows0, ss0)
    _wait_s(NCH - 1, rows0, ss0)

    plsc.subcore_barrier()
    pltpu.sync_copy(acc.at[pl.ds(sid * RPT, RPT), :],
                    out_hbm.at[cid, pl.ds(sid * RPT, RPT), :])


def _eg_body(e0_hbm, e1_hbm, h_hbm, uv_hbm, i0, i1,
             ru0, ru1, rv0, rv1, rp0, rp1, su0, su1, sv0, sv1, sw0, sw1):
    cid = lax.axis_index("c")
    sid = lax.axis_index("s")
    wid = sid * NC + cid
    base = wid * EPW
    pltpu.sync_copy(e0_hbm.at[pl.ds(base, EPW)], i0)
    pltpu.sync_copy(e1_hbm.at[pl.ds(base, EPW)], i1)

    def _gath(k, ru, su, rv, sv):
        pltpu.async_copy(h_hbm.at[i0.at[pl.ds(k * CH, CH)]], ru, su)
        pltpu.async_copy(h_hbm.at[i1.at[pl.ds(k * CH, CH)]], rv, sv)

    def _wait(k, ru, su, rv, sv):
        pltpu.make_async_copy(h_hbm.at[i0.at[pl.ds(k * CH, CH)]], ru, su).wait()
        pltpu.make_async_copy(h_hbm.at[i1.at[pl.ds(k * CH, CH)]], rv, sv).wait()

    def _pack(ru, rv, rp):
        def _rows8(r8, c):
            for dr in range(8):
                r = r8 * 8 + dr
                for c4 in range(HID // L):
                    rp[r, pl.ds(c4 * L, L)] = ru[r, pl.ds(c4 * L, L)]
                    rp[r, pl.ds(HID + c4 * L, L)] = rv[r, pl.ds(c4 * L, L)]
            return c
        lax.fori_loop(0, CH // 8, _rows8, 0)

    def _fire_w(k, rp, sw):
        pltpu.async_copy(rp, uv_hbm.at[pl.ds(base + k * CH, CH), :], sw)

    def _wait_w(k, rp, sw):
        pltpu.make_async_copy(rp, uv_hbm.at[pl.ds(base + k * CH, CH), :],
                              sw).wait()

    _gath(0, ru0, su0, rv0, sv0)

    def _pair(p, c):
        k0 = 2 * p
        k1 = k0 + 1
        _wait(k0, ru0, su0, rv0, sv0)
        _gath(k1, ru1, su1, rv1, sv1)

        @pl.when(p > 0)
        def _():
            _wait_w(k0 - 2, rp0, sw0)
        _pack(ru0, rv0, rp0)
        _fire_w(k0, rp0, sw0)

        _wait(k1, ru1, su1, rv1, sv1)

        @pl.when(k0 + 2 < NCH)
        def _():
            _gath(k0 + 2, ru0, su0, rv0, sv0)

        @pl.when(p > 0)
        def _():
            _wait_w(k1 - 2, rp1, sw1)
        _pack(ru1, rv1, rp1)
        _fire_w(k1, rp1, sw1)
        return c
    lax.fori_loop(0, NCH // 2, _pair, 0)

    _wait(NCH - 1, ru0, su0, rv0, sv0)
    _wait_w(NCH - 3, rp0, sw0)
    _pack(ru0, rv0, rp0)
    _fire_w(NCH - 1, rp0, sw0)
    _wait_w(NCH - 2, rp1, sw1)
    _wait_w(NCH - 1, rp0, sw0)


def _dis_col(degp):
    ones = jnp.ones((NW, 1), jnp.float32)
    deg = lax.dot_general(degp, ones, (((0,), (0,)), ((), ())),
                          preferred_element_type=jnp.float32)
    return lax.rsqrt(deg[:N] + 1.0)


def _pad_cols(a):
    return jnp.concatenate(
        [a, jnp.zeros((a.shape[0], HIDP - HID), jnp.float32)], axis=1)


def _prep1_body(degp_ref, x_ref, w_ref, y_ref):
    dis = _dis_col(degp_ref[...])
    y = jnp.dot(x_ref[...], w_ref[...],
                preferred_element_type=jnp.float32) * dis
    y_ref[...] = _pad_cols(y)


_prep1 = pl.pallas_call(
    _prep1_body,
    out_shape=jax.ShapeDtypeStruct((N, HIDP), jnp.float32),
)


def _mid_body(degp_ref, accp_ref, y1_ref, b1_ref, w2_ref, y2_ref):
    dis = _dis_col(degp_ref[...])
    acc = accp_ref[0, :N, 0:HID] + accp_ref[1, :N, 0:HID]
    h1 = jnp.maximum((acc + y1_ref[:, 0:HID]) * dis + b1_ref[...], 0.0)
    y2 = jnp.dot(h1, w2_ref[...], preferred_element_type=jnp.float32) * dis
    y2_ref[...] = _pad_cols(y2)


_mid = pl.pallas_call(
    _mid_body,
    out_shape=jax.ShapeDtypeStruct((N, HIDP), jnp.float32),
)


def _fin_body(degp_ref, accp_ref, y2_ref, b2_ref, h2_ref):
    dis = _dis_col(degp_ref[...])
    acc = accp_ref[0, :N, 0:HID] + accp_ref[1, :N, 0:HID]
    h2 = jnp.maximum((acc + y2_ref[:, 0:HID]) * dis + b2_ref[...], 0.0)
    h2_ref[...] = _pad_cols(h2)


_fin = pl.pallas_call(
    _fin_body,
    out_shape=jax.ShapeDtypeStruct((N, HIDP), jnp.float32),
)


BE = 16000


def _mlp_body(uv_ref, w1_ref, b1_ref, w2_ref, b2_ref, o_ref):
    bf = jnp.bfloat16
    uf = uv_ref[:, 0:HID]
    vf = uv_ref[:, HID:2 * HID]
    w1 = w1_ref[...].astype(bf)
    t = jnp.dot(uf.astype(bf), w1[0:HID], preferred_element_type=jnp.float32)
    t = t + jnp.dot(vf.astype(bf), w1[HID:2 * HID],
                    preferred_element_type=jnp.float32)
    t = t + jnp.dot(jnp.abs(uf - vf).astype(bf), w1[2 * HID:3 * HID],
                    preferred_element_type=jnp.float32)
    t = t + jnp.dot((uf * vf).astype(bf), w1[3 * HID:4 * HID],
                    preferred_element_type=jnp.float32)
    t = jnp.maximum(t + b1_ref[...], 0.0)
    o2 = jnp.dot(t.astype(bf), w2_ref[...].astype(bf),
                 preferred_element_type=jnp.float32)
    o_ref[...] = o2 + b2_ref[...]


_mlp = pl.pallas_call(
    _mlp_body,
    grid=(E // BE,),
    in_specs=[
        pl.BlockSpec((BE, HIDP), lambda i: (i, 0)),
        pl.BlockSpec((4 * HID, HID), lambda i: (0, 0)),
        pl.BlockSpec((HID,), lambda i: (0,)),
        pl.BlockSpec((HID, 1), lambda i: (0, 0)),
        pl.BlockSpec((1,), lambda i: (0,)),
    ],
    out_specs=pl.BlockSpec((BE, 1), lambda i: (i, 0)),
    out_shape=jax.ShapeDtypeStruct((E, 1), jnp.float32),
)


def kernel(x, graph_edge_index, edge_pairs, W_gcn1, b_gcn1, W_gcn2, b_gcn2,
           W_mlp1, b_mlp1, W_mlp2, b_mlp2):
    _deg_sc, _mp_sc, _eg_sc = _sc_kernels()
    src = graph_edge_index[0]
    dst = graph_edge_index[1]
    dst2 = dst.reshape(NW, NCH, CH)

    degp = _deg_sc(dst)
    y1 = _prep1(degp, x, W_gcn1)
    accp1 = _mp_sc(src, dst2, y1)
    y2 = _mid(degp, accp1, y1, b_gcn1, W_gcn2)
    accp2 = _mp_sc(src, dst2, y2)
    h2 = _fin(degp, accp2, y2, b_gcn2)
    uv = _eg_sc(edge_pairs[0], edge_pairs[1], h2)
    return _mlp(uv, W_mlp1, b_mlp1, W_mlp2, b_mlp2).reshape(-1)
